# baseline (speedup 1.0000x reference)
import jax
import jax.numpy as jnp
from jax import lax
from jax.experimental import pallas as pl
from jax.experimental.pallas import tpu as pltpu
from jax.experimental.pallas import tpu_sc as plsc

NC = 2
NS = 16
NW = NC * NS
L = 16

B = 16384
J = 20
EMB = 64

B_W = B // NW
CB = 32
NCH = B_W // CB
NEG_ROWS = CB * J
G = 128
NG = NEG_ROWS // G
NB_W = B_W * J
PAD = L


def _sgns_body(c_h, o_h, n_h, vemb, uemb, pos_h, negout_h,
               cidx, oidx, nidx, cg, og, ng, vrows, orows, nrows,
               posb, negb, sem):
    w = lax.axis_index("s") * NC + lax.axis_index("c")

    pltpu.sync_copy(c_h.at[w], cidx.at[pl.ds(0, B_W)])
    pltpu.sync_copy(o_h.at[w], oidx.at[pl.ds(0, B_W)])
    pltpu.sync_copy(n_h.at[w], nidx.at[pl.ds(0, NB_W)])

    lane = lax.iota(jnp.int32, L)
    last = lane == (L - 1)

    def shift_co(i, carry):
        cg[pl.ds(i * L, L)] = lax.shift_right_logical(cidx[pl.ds(i * L, L)], 1)
        og[pl.ds(i * L, L)] = lax.shift_right_logical(oidx[pl.ds(i * L, L)], 1)
        return carry

    def shift_n(i, carry):
        ng[pl.ds(i * L, L)] = lax.shift_right_logical(nidx[pl.ds(i * L, L)], 1)
        return carry

    lax.fori_loop(0, B_W // L, shift_co, 0)
    lax.fori_loop(0, NB_W // L, shift_n, 0)

    def chunk(ch, carry):
        cps = [
            pltpu.async_copy(vemb.at[cg.at[pl.ds(ch * CB, CB)]], vrows, sem),
            pltpu.async_copy(uemb.at[og.at[pl.ds(ch * CB, CB)]], orows, sem),
        ]
        for k in range(NG):
            cps.append(pltpu.async_copy(
                uemb.at[ng.at[pl.ds((ch * NG + k) * G, G)]],
                nrows.at[pl.ds(k * G, G)], sem))
        for cp in cps:
            cp.wait()

        def bbody(bl, c2):
            fb = ch * CB + bl
            co_vec = cidx[pl.ds(fb, L)]
            oo_vec = oidx[pl.ds(fb, L)]
            voff = (co_vec[0] & 1) * EMB
            v0 = vrows[bl, pl.ds(voff, L)]
            v1 = vrows[bl, pl.ds(voff + L, L)]
            v2 = vrows[bl, pl.ds(voff + 2 * L, L)]
            v3 = vrows[bl, pl.ds(voff + 3 * L, L)]

            def dot_store(rref, row, off, pos):
                acc = rref[row, pl.ds(off, L)] * v0
                acc = acc + rref[row, pl.ds(off + L, L)] * v1
                acc = acc + rref[row, pl.ds(off + 2 * L, L)] * v2
                acc = acc + rref[row, pl.ds(off + 3 * L, L)] * v3
                s = plsc.cumsum(acc)
                idx = jnp.full((L,), pos, dtype=jnp.int32)
                plsc.store_scatter(negb, [idx], s, mask=last)

            acc = orows[bl, pl.ds((oo_vec[0] & 1) * EMB, L)] * v0
            po = (oo_vec[0] & 1) * EMB
            acc = acc + orows[bl, pl.ds(po + L, L)] * v1
            acc = acc + orows[bl, pl.ds(po + 2 * L, L)] * v2
            acc = acc + orows[bl, pl.ds(po + 3 * L, L)] * v3
            s = plsc.cumsum(acc)
            idx = jnp.full((L,), fb, dtype=jnp.int32)
            plsc.store_scatter(posb, [idx], s, mask=last)

            fnb = fb * J
            pv0 = nidx[pl.ds(fnb, L)]
            pv1 = nidx[pl.ds(fnb + 4, L)]
            for j in range(J):
                n_orig = pv0[j] if j < L else pv1[j - 4]
                dot_store(nrows, bl * J + j, (n_orig & 1) * EMB, fnb + j)
            return c2

        lax.fori_loop(0, CB, bbody, 0)
        return carry

    lax.fori_loop(0, NCH, chunk, 0)

    def sig_pos(i, c2):
        x = posb[pl.ds(i * L, L)]
        posb[pl.ds(i * L, L)] = 1.0 / (1.0 + jnp.exp(-x))
        return c2

    def sig_neg(i, c2):
        x = negb[pl.ds(i * L, L)]
        negb[pl.ds(i * L, L)] = 1.0 / (1.0 + jnp.exp(x))
        return c2

    lax.fori_loop(0, B_W // L, sig_pos, 0)
    lax.fori_loop(0, NB_W // L, sig_neg, 0)

    pltpu.sync_copy(posb, pos_h.at[w])
    pltpu.sync_copy(negb, negout_h.at[w])


@jax.jit
def _sgns(c_h, o_h, n_h, vemb, uemb):
    mesh = plsc.VectorSubcoreMesh(core_axis_name="c", subcore_axis_name="s",
                                  num_cores=NC, num_subcores=NS)
    f = pl.kernel(
        _sgns_body,
        out_type=(
            jax.ShapeDtypeStruct((NW, B_W), jnp.float32),
            jax.ShapeDtypeStruct((NW, NB_W), jnp.float32),
        ),
        mesh=mesh,
        scratch_types=[
            pltpu.VMEM((B_W + PAD,), jnp.int32),
            pltpu.VMEM((B_W + PAD,), jnp.int32),
            pltpu.VMEM((NB_W + PAD,), jnp.int32),
            pltpu.VMEM((B_W,), jnp.int32),
            pltpu.VMEM((B_W,), jnp.int32),
            pltpu.VMEM((NB_W,), jnp.int32),
            pltpu.VMEM((CB, 2 * EMB), jnp.float32),
            pltpu.VMEM((CB, 2 * EMB), jnp.float32),
            pltpu.VMEM((NEG_ROWS, 2 * EMB), jnp.float32),
            pltpu.VMEM((B_W,), jnp.float32),
            pltpu.VMEM((NB_W,), jnp.float32),
            pltpu.SemaphoreType.DMA,
        ],
        compiler_params=pltpu.CompilerParams(needs_layout_passes=False,
                                             use_tc_tiling_on_sc=True),
    )
    return f(c_h, o_h, n_h, vemb, uemb)


def kernel(c, o, neg, vEmbedding, uEmbedding):
    c_h = c.reshape(NW, B_W).astype(jnp.int32)
    o_h = o.reshape(NW, B_W).astype(jnp.int32)
    n_h = neg.reshape(NW, NB_W).astype(jnp.int32)
    v2 = vEmbedding.reshape(vEmbedding.shape[0] // 2, 2 * EMB)
    u2 = uEmbedding.reshape(uEmbedding.shape[0] // 2, 2 * EMB)
    pos, negout = _sgns(c_h, o_h, n_h, v2, u2)
    return pos.reshape(B, 1), negout.reshape(B, J, 1)
